# baseline (speedup 1.0000x reference)
import functools

import jax
import jax.numpy as jnp
from jax import lax
from jax.experimental import pallas as pl
from jax.experimental.pallas import tpu as pltpu
from jax.experimental.pallas import tpu_sc as plsc

_N = 10000
_E = 160000
_D = 256
_C = 64
_PAIR_CHUNKS = 168
_CF = 110
_CS = _PAIR_CHUNKS - _CF
_TOT = 16 * _PAIR_CHUNKS * _C


@functools.partial(
    pl.kernel,
    mesh=plsc.VectorSubcoreMesh(core_axis_name="c", subcore_axis_name="s"),
    out_type=jax.ShapeDtypeStruct((_TOT, _D), jnp.float32),
    scratch_types=[
        pltpu.VMEM((_CF * _C,), jnp.int32),
        pltpu.VMEM((_CF * _C,), jnp.int32),
        pltpu.VMEM((_C, _D), jnp.float32),
        pltpu.VMEM((_C, _D), jnp.float32),
        pltpu.VMEM((_C, _D), jnp.float32),
        pltpu.VMEM((_C, _D), jnp.float32),
        pltpu.VMEM((_C, _D), jnp.float32),
        pltpu.VMEM((_C, _D), jnp.float32),
        pltpu.SemaphoreType.DMA,
        pltpu.SemaphoreType.DMA,
        pltpu.SemaphoreType.DMA,
        pltpu.SemaphoreType.DMA,
    ],
)
def _unpool_kernel(table, idx0, idx1, out, idx0_v, idx1_v,
                   rows0a, rows1a, rows0b, rows1b, outa, outb,
                   gsema, gsemb, ssema, ssemb):
    c = lax.axis_index("c")
    s = lax.axis_index("s")
    is_fast = c == 0
    nchunk = jnp.where(is_fast, _CF, _CS)
    base = s * (_PAIR_CHUNKS * _C) + jnp.where(is_fast, 0, _CF * _C)

    pltpu.sync_copy(idx0.at[pl.ds(base, _CS * _C)], idx0_v.at[pl.ds(0, _CS * _C)])
    pltpu.sync_copy(idx1.at[pl.ds(base, _CS * _C)], idx1_v.at[pl.ds(0, _CS * _C)])

    @pl.when(is_fast)
    def _():
        rest = (_CF - _CS) * _C
        pltpu.sync_copy(idx0.at[pl.ds(base + _CS * _C, rest)],
                        idx0_v.at[pl.ds(_CS * _C, rest)])
        pltpu.sync_copy(idx1.at[pl.ds(base + _CS * _C, rest)],
                        idx1_v.at[pl.ds(_CS * _C, rest)])

    sets = ((rows0a, rows1a, outa, gsema, ssema),
            (rows0b, rows1b, outb, gsemb, ssemb))

    def gathers(b, g):
        rows0, rows1, _, gsem, _ = sets[b]
        c0 = pltpu.make_async_copy(
            table.at[idx0_v.at[pl.ds(g * _C, _C)]], rows0, gsem)
        c1 = pltpu.make_async_copy(
            table.at[idx1_v.at[pl.ds(g * _C, _C)]], rows1, gsem)
        return c0, c1

    def store(b, g):
        _, _, stg, _, ssem = sets[b]
        return pltpu.make_async_copy(
            stg, out.at[pl.ds(base + g * _C, _C)], ssem)

    for b in range(2):
        c0, c1 = gathers(b, b)
        c0.start()
        c1.start()

    def chunk_step(g, carry):
        for b in range(2):
            @pl.when(g % 2 == b)
            def _():
                rows0, rows1, stg, _, _ = sets[b]
                c0, c1 = gathers(b, g)
                c0.wait()
                c1.wait()

                @pl.when(g >= 2)
                def _():
                    store(b, g - 2).wait()

                def row(r, c2):
                    for j in range(_D // 16):
                        sl = pl.ds(j * 16, 16)
                        stg[r, sl] = (rows0[r, sl] + rows1[r, sl]) * 0.5
                    return c2

                lax.fori_loop(0, _C, row, 0)
                store(b, g).start()

                @pl.when(g + 2 < nchunk)
                def _():
                    n0, n1 = gathers(b, g + 2)
                    n0.start()
                    n1.start()
        return carry

    lax.fori_loop(0, nchunk, chunk_step, 0)

    store(0, 0).wait()
    store(1, 0).wait()


def kernel(inputs, unpool_idx):
    table = inputs.reshape(_N, _D)
    idx = unpool_idx.astype(jnp.int32)
    self_ids = jnp.arange(_N, dtype=jnp.int32)
    pad = jnp.zeros((_TOT - _N - _E,), jnp.int32)
    idx0 = jnp.concatenate([self_ids, idx[:, 0], pad])
    idx1 = jnp.concatenate([self_ids, idx[:, 1], pad])
    out = _unpool_kernel(table, idx0, idx1)
    return out[None, : _N + _E, :]


# speedup vs baseline: 1.0011x; 1.0011x over previous
import functools

import jax
import jax.numpy as jnp
from jax import lax
from jax.experimental import pallas as pl
from jax.experimental.pallas import tpu as pltpu
from jax.experimental.pallas import tpu_sc as plsc

_N = 10000
_E = 160000
_D = 256
_C = 64
_PAIR_CHUNKS = 168
_CF = 110
_CS = _PAIR_CHUNKS - _CF
_TOT = 16 * _PAIR_CHUNKS * _C


@functools.partial(
    pl.kernel,
    mesh=plsc.VectorSubcoreMesh(core_axis_name="c", subcore_axis_name="s"),
    out_type=jax.ShapeDtypeStruct((_TOT, _D), jnp.float32),
    scratch_types=[
        pltpu.VMEM((_CF * _C,), jnp.int32),
        pltpu.VMEM((_CF * _C,), jnp.int32),
        pltpu.VMEM((_C, _D), jnp.float32),
        pltpu.VMEM((_C, _D), jnp.float32),
        pltpu.VMEM((_C, _D), jnp.float32),
        pltpu.VMEM((_C, _D), jnp.float32),
        pltpu.VMEM((_C, _D), jnp.float32),
        pltpu.VMEM((_C, _D), jnp.float32),
        pltpu.SemaphoreType.DMA,
        pltpu.SemaphoreType.DMA,
        pltpu.SemaphoreType.DMA,
        pltpu.SemaphoreType.DMA,
    ],
)
def _unpool_kernel(table, idx0, idx1, out, idx0_v, idx1_v,
                   rows0a, rows1a, rows0b, rows1b, outa, outb,
                   gsema, gsemb, ssema, ssemb):
    c = lax.axis_index("c")
    s = lax.axis_index("s")
    is_fast = s < 8
    pair = lax.rem(s, 8) * 2 + c
    nchunk = jnp.where(is_fast, _CF, _CS)
    base = pair * (_PAIR_CHUNKS * _C) + jnp.where(is_fast, 0, _CF * _C)

    pltpu.sync_copy(idx0.at[pl.ds(base, _CS * _C)], idx0_v.at[pl.ds(0, _CS * _C)])
    pltpu.sync_copy(idx1.at[pl.ds(base, _CS * _C)], idx1_v.at[pl.ds(0, _CS * _C)])

    @pl.when(is_fast)
    def _():
        rest = (_CF - _CS) * _C
        pltpu.sync_copy(idx0.at[pl.ds(base + _CS * _C, rest)],
                        idx0_v.at[pl.ds(_CS * _C, rest)])
        pltpu.sync_copy(idx1.at[pl.ds(base + _CS * _C, rest)],
                        idx1_v.at[pl.ds(_CS * _C, rest)])

    sets = ((rows0a, rows1a, outa, gsema, ssema),
            (rows0b, rows1b, outb, gsemb, ssemb))

    def gathers(b, g):
        rows0, rows1, _, gsem, _ = sets[b]
        c0 = pltpu.make_async_copy(
            table.at[idx0_v.at[pl.ds(g * _C, _C)]], rows0, gsem)
        c1 = pltpu.make_async_copy(
            table.at[idx1_v.at[pl.ds(g * _C, _C)]], rows1, gsem)
        return c0, c1

    def store(b, g):
        _, _, stg, _, ssem = sets[b]
        return pltpu.make_async_copy(
            stg, out.at[pl.ds(base + g * _C, _C)], ssem)

    for b in range(2):
        c0, c1 = gathers(b, b)
        c0.start()
        c1.start()

    def chunk_step(g, carry):
        for b in range(2):
            @pl.when(g % 2 == b)
            def _():
                rows0, rows1, stg, _, _ = sets[b]
                c0, c1 = gathers(b, g)
                c0.wait()
                c1.wait()

                @pl.when(g >= 2)
                def _():
                    store(b, g - 2).wait()

                def row(r, c2):
                    for j in range(_D // 16):
                        sl = pl.ds(j * 16, 16)
                        stg[r, sl] = (rows0[r, sl] + rows1[r, sl]) * 0.5
                    return c2

                lax.fori_loop(0, _C, row, 0)
                store(b, g).start()

                @pl.when(g + 2 < nchunk)
                def _():
                    n0, n1 = gathers(b, g + 2)
                    n0.start()
                    n1.start()
        return carry

    lax.fori_loop(0, nchunk, chunk_step, 0)

    store(0, 0).wait()
    store(1, 0).wait()


def kernel(inputs, unpool_idx):
    table = inputs.reshape(_N, _D)
    idx = unpool_idx.astype(jnp.int32)
    self_ids = jnp.arange(_N, dtype=jnp.int32)
    pad = jnp.zeros((_TOT - _N - _E,), jnp.int32)
    idx0 = jnp.concatenate([self_ids, idx[:, 0], pad])
    idx1 = jnp.concatenate([self_ids, idx[:, 1], pad])
    out = _unpool_kernel(table, idx0, idx1)
    return out[None, : _N + _E, :]


# speedup vs baseline: 2.0473x; 2.0451x over previous
import functools

import jax
import jax.numpy as jnp
from jax import lax
from jax.experimental import pallas as pl
from jax.experimental.pallas import tpu as pltpu
from jax.experimental.pallas import tpu_sc as plsc

_N = 10000
_E = 160000
_D = 256
_NW = 32
_C = 64
_CPW = 84
_NCID = _NW * _CPW
_LAST = _N + _E - _C


@functools.partial(
    pl.kernel,
    mesh=plsc.VectorSubcoreMesh(core_axis_name="c", subcore_axis_name="s"),
    out_type=jax.ShapeDtypeStruct((_N + _E, _D), jnp.float32),
    scratch_types=[
        pltpu.VMEM((2 * _C,), jnp.int32),
        pltpu.VMEM((2 * _C,), jnp.int32),
        pltpu.VMEM((_C, _D), jnp.float32),
        pltpu.VMEM((_C, _D), jnp.float32),
        pltpu.VMEM((_C, _D), jnp.float32),
        pltpu.VMEM((_C, _D), jnp.float32),
        pltpu.VMEM((_C, _D), jnp.float32),
        pltpu.VMEM((_C, _D), jnp.float32),
        pltpu.SemaphoreType.DMA,
        pltpu.SemaphoreType.DMA,
        pltpu.SemaphoreType.DMA,
        pltpu.SemaphoreType.DMA,
        pltpu.SemaphoreType.DMA,
        pltpu.SemaphoreType.DMA,
    ],
)
def _unpool_kernel(table, idxmat, out, idxa, idxb,
                   rows0a, rows1a, rows0b, rows1b, stga, stgb,
                   isema, isemb, gsema, gsemb, ssema, ssemb):
    w = lax.axis_index("s") * 2 + lax.axis_index("c")

    sets = ((idxa, rows0a, rows1a, stga, isema, gsema, ssema),
            (idxb, rows0b, rows1b, stgb, isemb, gsemb, ssemb))

    def idxload(b, g):
        idxv, _, _, _, isem, _, _ = sets[b]
        cid = w + g * _NW
        return pltpu.make_async_copy(
            idxmat.at[pl.ds(cid * (2 * _C), 2 * _C)], idxv, isem)

    def gathers(b):
        idxv, rows0, rows1, _, _, gsem, _ = sets[b]
        c0 = pltpu.make_async_copy(
            table.at[idxv.at[pl.ds(0, _C)]], rows0, gsem)
        c1 = pltpu.make_async_copy(
            table.at[idxv.at[pl.ds(_C, _C)]], rows1, gsem)
        return c0, c1

    def store(b, g):
        _, _, _, stg, _, _, ssem = sets[b]
        cid = w + g * _NW
        base = jnp.minimum(cid * _C, _LAST)
        return pltpu.make_async_copy(stg, out.at[pl.ds(base, _C)], ssem)

    for b in range(2):
        idxload(b, b).start()
    for b in range(2):
        idxload(b, b).wait()
        c0, c1 = gathers(b)
        c0.start()
        c1.start()

    def chunk_step(g, carry):
        for b in range(2):
            @pl.when(g % 2 == b)
            def _():
                _, rows0, rows1, stg, _, _, _ = sets[b]
                c0, c1 = gathers(b)
                c0.wait()
                c1.wait()

                @pl.when(g + 2 < _CPW)
                def _():
                    idxload(b, g + 2).start()

                @pl.when(g >= 2)
                def _():
                    store(b, g - 2).wait()

                def row(r, c2):
                    for j in range(_D // 16):
                        sl = pl.ds(j * 16, 16)
                        stg[r, sl] = (rows0[r, sl] + rows1[r, sl]) * 0.5
                    return c2

                lax.fori_loop(0, _C, row, 0)
                store(b, g).start()

                @pl.when(g + 2 < _CPW)
                def _():
                    idxload(b, g + 2).wait()
                    n0, n1 = gathers(b)
                    n0.start()
                    n1.start()
        return carry

    lax.fori_loop(0, _CPW, chunk_step, 0)

    store(0, 0).wait()
    store(1, 0).wait()


def kernel(inputs, unpool_idx):
    table = inputs.reshape(_N, _D)
    idx = unpool_idx.astype(jnp.int32)
    self_ids = jnp.arange(_N, dtype=jnp.int32)
    idx0 = jnp.concatenate([self_ids, idx[:, 0]])
    idx1 = jnp.concatenate([self_ids, idx[:, 1]])
    starts = jnp.minimum(jnp.arange(_NCID, dtype=jnp.int32) * _C, _LAST)
    pos = starts[:, None] + jnp.arange(_C, dtype=jnp.int32)[None, :]
    idxmat = jnp.concatenate([idx0[pos], idx1[pos]], axis=1).reshape(-1)
    out = _unpool_kernel(table, idxmat)
    return out[None]


# speedup vs baseline: 2.5048x; 1.2235x over previous
import functools

import jax
import jax.numpy as jnp
from jax import lax
from jax.experimental import pallas as pl
from jax.experimental.pallas import tpu as pltpu
from jax.experimental.pallas import tpu_sc as plsc

_N = 10000
_E = 160000
_D = 256
_NW = 32
_C = 64
_CPW = 84
_NCID = _NW * _CPW
_LAST = _N + _E - _C


@functools.partial(
    pl.kernel,
    mesh=plsc.VectorSubcoreMesh(core_axis_name="c", subcore_axis_name="s"),
    out_type=jax.ShapeDtypeStruct((_N + _E, _D), jnp.float32),
    scratch_types=[
        pltpu.VMEM((2 * _C,), jnp.int32),
        pltpu.VMEM((2 * _C,), jnp.int32),
        pltpu.VMEM((_C, _D), jnp.float32),
        pltpu.VMEM((_C, _D), jnp.float32),
        pltpu.VMEM((_C, _D), jnp.float32),
        pltpu.VMEM((_C, _D), jnp.float32),
        pltpu.VMEM((_C, _D), jnp.float32),
        pltpu.VMEM((_C, _D), jnp.float32),
        pltpu.SemaphoreType.DMA,
        pltpu.SemaphoreType.DMA,
        pltpu.SemaphoreType.DMA,
        pltpu.SemaphoreType.DMA,
        pltpu.SemaphoreType.DMA,
        pltpu.SemaphoreType.DMA,
    ],
)
def _unpool_kernel(table, idx0, idx1, out, idxa, idxb,
                   rows0a, rows1a, rows0b, rows1b, stga, stgb,
                   isema, isemb, gsema, gsemb, ssema, ssemb):
    w = lax.axis_index("s") * 2 + lax.axis_index("c")

    sets = ((idxa, rows0a, rows1a, stga, isema, gsema, ssema),
            (idxb, rows0b, rows1b, stgb, isemb, gsemb, ssemb))

    def idxload(b, g):
        idxv, _, _, _, isem, _, _ = sets[b]
        cid = w + g * _NW
        base = jnp.minimum(cid * _C, _LAST)
        c0 = pltpu.make_async_copy(
            idx0.at[pl.ds(base, _C)], idxv.at[pl.ds(0, _C)], isem)
        c1 = pltpu.make_async_copy(
            idx1.at[pl.ds(base, _C)], idxv.at[pl.ds(_C, _C)], isem)
        return c0, c1

    def gathers(b):
        idxv, rows0, rows1, _, _, gsem, _ = sets[b]
        c0 = pltpu.make_async_copy(
            table.at[idxv.at[pl.ds(0, _C)]], rows0, gsem)
        c1 = pltpu.make_async_copy(
            table.at[idxv.at[pl.ds(_C, _C)]], rows1, gsem)
        return c0, c1

    def store(b, g):
        _, _, _, stg, _, _, ssem = sets[b]
        cid = w + g * _NW
        base = jnp.minimum(cid * _C, _LAST)
        return pltpu.make_async_copy(stg, out.at[pl.ds(base, _C)], ssem)

    for b in range(2):
        i0, i1 = idxload(b, b)
        i0.start()
        i1.start()
    for b in range(2):
        i0, i1 = idxload(b, b)
        i0.wait()
        i1.wait()
        c0, c1 = gathers(b)
        c0.start()
        c1.start()

    def chunk_step(g, carry):
        for b in range(2):
            @pl.when(g % 2 == b)
            def _():
                _, rows0, rows1, stg, _, _, _ = sets[b]
                c0, c1 = gathers(b)
                c0.wait()
                c1.wait()

                @pl.when(g + 2 < _CPW)
                def _():
                    i0, i1 = idxload(b, g + 2)
                    i0.start()
                    i1.start()

                @pl.when(g >= 2)
                def _():
                    store(b, g - 2).wait()

                def row(r, c2):
                    for j in range(_D // 16):
                        sl = pl.ds(j * 16, 16)
                        stg[r, sl] = (rows0[r, sl] + rows1[r, sl]) * 0.5
                    return c2

                lax.fori_loop(0, _C, row, 0)
                store(b, g).start()

                @pl.when(g + 2 < _CPW)
                def _():
                    i0, i1 = idxload(b, g + 2)
                    i0.wait()
                    i1.wait()
                    n0, n1 = gathers(b)
                    n0.start()
                    n1.start()
        return carry

    lax.fori_loop(0, _CPW, chunk_step, 0)

    store(0, 0).wait()
    store(1, 0).wait()


def kernel(inputs, unpool_idx):
    table = inputs.reshape(_N, _D)
    idx = unpool_idx.astype(jnp.int32)
    self_ids = jnp.arange(_N, dtype=jnp.int32)
    idx0 = jnp.concatenate([self_ids, idx[:, 0]])
    idx1 = jnp.concatenate([self_ids, idx[:, 1]])
    out = _unpool_kernel(table, idx0, idx1)
    return out[None]
